# Initial kernel scaffold; baseline (speedup 1.0000x reference)
import jax
import jax.numpy as jnp
from jax.experimental import pallas as pl


def kernel(x, edge_index, batch, descriptors, W1, b1, g1, be1, convWb, convWc, convbc, convbias, convg, convbe, Wm1, gm1, bm1, Wm2, gm2, bm2, W2, b2, g3, be3, Wout, bout):
    raise NotImplementedError("write your pallas kernel here")


# speedup vs baseline: 1.0223x; 1.0223x over previous
import functools

import jax
import jax.numpy as jnp
import numpy as np
from jax import lax
from jax.experimental import pallas as pl
from jax.experimental.pallas import tpu as pltpu
from jax.experimental.pallas import tpu_sc as plsc

N = 10000
NPAD = 10240
HID = 128
LAYERS = 4
HEADS = 8
BASES = 4
F = HID // HEADS
DESC = 200
NGRAPH = 128

NC = 2
NS = 16
NW = NC * NS
CH = 128


def _sc_degree_kernel(epw_chunks):
    mesh = plsc.VectorSubcoreMesh(core_axis_name="c", subcore_axis_name="s")
    rows_per_s = NPAD // NS

    @functools.partial(
        pl.kernel,
        mesh=mesh,
        out_type=jax.ShapeDtypeStruct((NC, NPAD), jnp.float32),
        scratch_types=[
            pltpu.VMEM((epw_chunks, CH), jnp.int32),
            pltpu.VMEM((CH,), jnp.float32),
            pltpu.VMEM((rows_per_s,), jnp.float32),
            pltpu.VMEM_SHARED((NPAD,), jnp.float32),
        ],
    )
    def k(dst_hbm, out_hbm, idx_v, ones_v, zeros_v, cnt_sh):
        c = lax.axis_index("c")
        s = lax.axis_index("s")
        wid = s * NC + c
        for i in range(CH // 16):
            ones_v[pl.ds(i * 16, 16)] = jnp.ones((16,), jnp.float32)

        def zbody(i, carry):
            zeros_v[pl.ds(i * 16, 16)] = jnp.zeros((16,), jnp.float32)
            return carry

        lax.fori_loop(0, rows_per_s // 16, zbody, 0)
        pltpu.sync_copy(zeros_v, cnt_sh.at[pl.ds(s * rows_per_s, rows_per_s)])
        pltpu.sync_copy(dst_hbm.at[wid], idx_v)
        plsc.subcore_barrier()

        def body(j, carry):
            pltpu.sync_copy(ones_v, cnt_sh.at[idx_v.at[j]], add=True)
            return carry

        lax.fori_loop(0, epw_chunks, body, 0)
        plsc.subcore_barrier()
        pltpu.sync_copy(
            cnt_sh.at[pl.ds(s * rows_per_s, rows_per_s)],
            out_hbm.at[c, pl.ds(s * rows_per_s, rows_per_s)],
        )

    return k


def _pad_edges(idx, total):
    pad = total - idx.shape[0]
    pad_rows = jnp.asarray(N + (np.arange(pad) % (NPAD - N)), jnp.int32)
    return jnp.concatenate([idx, pad_rows])


def _bn(x, g, b, eps=1e-5):
    mu = jnp.mean(x, axis=0)
    var = jnp.var(x, axis=0)
    return (x - mu) * lax.rsqrt(var + eps) * g + b


def kernel(x, edge_index, batch, descriptors, W1, b1, g1, be1, convWb, convWc,
           convbc, convbias, convg, convbe, Wm1, gm1, bm1, Wm2, gm2, bm2, W2,
           b2, g3, be3, Wout, bout):
    n = N
    E = edge_index.shape[1]
    epd = ((E + NW * CH - 1) // (NW * CH)) * (NW * CH)
    dst3d = _pad_edges(edge_index[1], epd).reshape(NW, epd // (NW * CH), CH)
    parts = _sc_degree_kernel(epd // (NW * CH))(dst3d)
    deg = 1.0 + parts[0, :N] + parts[1, :N]
    dinv = lax.rsqrt(deg)

    loop = jnp.arange(n, dtype=edge_index.dtype)
    src = jnp.concatenate([edge_index[0], loop])
    dst = jnp.concatenate([edge_index[1], loop])
    w = dinv[src] * dinv[dst]
    h = jax.nn.relu(_bn(x @ W1 + b1, g1, be1))
    for l in range(LAYERS):
        bases = h @ convWb[l]
        wt = (h @ convWc[l] + convbc[l]).reshape(n, HEADS, BASES)
        agg = jnp.zeros((n, BASES * F), h.dtype).at[dst].add(w[:, None] * bases[src])
        agg = agg.reshape(n, BASES, F)
        o = jnp.einsum('nhb,nbf->nhf', wt, agg).reshape(n, HID) + convbias[l]
        h = h + jax.nn.relu(_bn(o, convg[l], convbe[l]))
    ssum = jax.ops.segment_sum(h, batch, num_segments=NGRAPH)
    cnt = jax.ops.segment_sum(jnp.ones((n,), h.dtype), batch, num_segments=NGRAPH)
    pooled = ssum / jnp.maximum(cnt, 1.0)[:, None]
    m = jax.nn.relu(_bn(pooled @ Wm1, gm1, bm1))
    m = jax.nn.relu(_bn(m @ Wm2, gm2, bm2))
    z = jnp.concatenate([m, descriptors], axis=1)
    z = jax.nn.relu(z @ W2 + b2)
    z = _bn(z, g3, be3)
    return z @ Wout + bout


# speedup vs baseline: 13.0066x; 12.7229x over previous
import functools

import jax
import jax.numpy as jnp
import numpy as np
from jax import lax
from jax.experimental import pallas as pl
from jax.experimental.pallas import tpu as pltpu
from jax.experimental.pallas import tpu_sc as plsc

N = 10000
NPAD = 10240
HID = 128
LAYERS = 4
HEADS = 8
BASES = 4
F = HID // HEADS
DESC = 200
NGRAPH = 128

NC = 2
NS = 16
NW = NC * NS
CH = 128


def _sc_degree_kernel(epw_chunks):
    mesh = plsc.VectorSubcoreMesh(core_axis_name="c", subcore_axis_name="s")
    rows_per_s = NPAD // NS

    @functools.partial(
        pl.kernel,
        mesh=mesh,
        out_type=jax.ShapeDtypeStruct((NC, NPAD), jnp.float32),
        scratch_types=[
            pltpu.VMEM((epw_chunks, CH), jnp.int32),
            pltpu.VMEM((CH,), jnp.float32),
            pltpu.VMEM((rows_per_s,), jnp.float32),
            pltpu.VMEM_SHARED((NPAD,), jnp.float32),
        ],
    )
    def k(dst_hbm, out_hbm, idx_v, ones_v, zeros_v, cnt_sh):
        c = lax.axis_index("c")
        s = lax.axis_index("s")
        wid = s * NC + c
        for i in range(CH // 16):
            ones_v[pl.ds(i * 16, 16)] = jnp.ones((16,), jnp.float32)

        def zbody(i, carry):
            zeros_v[pl.ds(i * 16, 16)] = jnp.zeros((16,), jnp.float32)
            return carry

        lax.fori_loop(0, rows_per_s // 16, zbody, 0)
        pltpu.sync_copy(zeros_v, cnt_sh.at[pl.ds(s * rows_per_s, rows_per_s)])
        pltpu.sync_copy(dst_hbm.at[wid], idx_v)
        plsc.subcore_barrier()

        def body(j, carry):
            pltpu.sync_copy(ones_v, cnt_sh.at[idx_v.at[j]], add=True)
            return carry

        lax.fori_loop(0, epw_chunks, body, 0)
        plsc.subcore_barrier()
        pltpu.sync_copy(
            cnt_sh.at[pl.ds(s * rows_per_s, rows_per_s)],
            out_hbm.at[c, pl.ds(s * rows_per_s, rows_per_s)],
        )

    return k


def _sc_layer_kernel(epw_chunks):
    mesh = plsc.VectorSubcoreMesh(core_axis_name="c", subcore_axis_name="s")
    rows_per_s = NPAD // NS

    @functools.partial(
        pl.kernel,
        mesh=mesh,
        out_type=jax.ShapeDtypeStruct((NC, NPAD, BASES * F), jnp.float32),
        compiler_params=pltpu.CompilerParams(use_tc_tiling_on_sc=False),
        scratch_types=[
            pltpu.VMEM((epw_chunks, CH), jnp.int32),
            pltpu.VMEM((epw_chunks, CH), jnp.int32),
            pltpu.VMEM((CH, BASES * F), jnp.float32),
            pltpu.VMEM((CH, BASES * F), jnp.float32),
            pltpu.VMEM_SHARED((NPAD, BASES * F), jnp.float32),
            pltpu.SemaphoreType.DMA,
        ],
    )
    def k(src_hbm, dst_hbm, bases_hbm, out_hbm, sidx_v, didx_v, zeros_v,
          rows_v, agg_sh, sem):
        c = lax.axis_index("c")
        s = lax.axis_index("s")
        wid = s * NC + c

        def zb(i, carry):
            def zb2(j, carry2):
                zeros_v[i, pl.ds(j * 16, 16)] = jnp.zeros((16,), jnp.float32)
                return carry2
            return lax.fori_loop(0, (BASES * F) // 16, zb2, carry)

        lax.fori_loop(0, CH, zb, 0)
        def zcopy(i, carry):
            pltpu.sync_copy(zeros_v, agg_sh.at[pl.ds(s * rows_per_s + i * CH, CH)])
            return carry

        lax.fori_loop(0, rows_per_s // CH, zcopy, 0)
        pltpu.sync_copy(src_hbm.at[wid], sidx_v)
        pltpu.sync_copy(dst_hbm.at[wid], didx_v)
        plsc.subcore_barrier()

        def body(j, carry):
            pltpu.async_copy(bases_hbm.at[sidx_v.at[j]], rows_v, sem).wait()
            pltpu.sync_copy(rows_v, agg_sh.at[didx_v.at[j]], add=True)
            return carry

        lax.fori_loop(0, epw_chunks, body, 0)
        plsc.subcore_barrier()
        pltpu.sync_copy(
            agg_sh.at[pl.ds(s * rows_per_s, rows_per_s)],
            out_hbm.at[c, pl.ds(s * rows_per_s, rows_per_s)],
        )

    return k


def _pad_edges(idx, total):
    pad = total - idx.shape[0]
    pad_rows = jnp.asarray(N + (np.arange(pad) % (NPAD - N)), jnp.int32)
    return jnp.concatenate([idx, pad_rows])


def _bn(x, g, b, eps=1e-5):
    mu = jnp.mean(x, axis=0)
    var = jnp.var(x, axis=0)
    return (x - mu) * lax.rsqrt(var + eps) * g + b


def kernel(x, edge_index, batch, descriptors, W1, b1, g1, be1, convWb, convWc,
           convbc, convbias, convg, convbe, Wm1, gm1, bm1, Wm2, gm2, bm2, W2,
           b2, g3, be3, Wout, bout):
    n = N
    E = edge_index.shape[1]
    epd = ((E + NW * CH - 1) // (NW * CH)) * (NW * CH)
    dst3d = _pad_edges(edge_index[1], epd).reshape(NW, epd // (NW * CH), CH)
    parts = _sc_degree_kernel(epd // (NW * CH))(dst3d)
    deg = 1.0 + parts[0, :N] + parts[1, :N]
    dinv = lax.rsqrt(deg)

    loop = jnp.arange(n, dtype=edge_index.dtype)
    etot = E + n
    ep = ((etot + NW * CH - 1) // (NW * CH)) * (NW * CH)
    npad_e = ep - etot
    src_pad = jnp.asarray((np.arange(npad_e) * 61) % N, jnp.int32)
    dst_pad = jnp.asarray(N + (np.arange(npad_e) % (NPAD - N)), jnp.int32)
    epw_chunks = ep // (NW * CH)
    src3d = jnp.concatenate([edge_index[0], loop, src_pad]).reshape(NW, epw_chunks, CH)
    dst3d = jnp.concatenate([edge_index[1], loop, dst_pad]).reshape(NW, epw_chunks, CH)
    layer_scatter = _sc_layer_kernel(epw_chunks)

    h = jax.nn.relu(_bn(x @ W1 + b1, g1, be1))
    for l in range(LAYERS):
        bases = dinv[:, None] * (h @ convWb[l])
        wt = (h @ convWc[l] + convbc[l]).reshape(n, HEADS, BASES)
        ps = layer_scatter(src3d, dst3d, bases)
        agg = (dinv[:, None] * (ps[0, :N] + ps[1, :N])).reshape(n, BASES, F)
        o = jnp.einsum('nhb,nbf->nhf', wt, agg).reshape(n, HID) + convbias[l]
        h = h + jax.nn.relu(_bn(o, convg[l], convbe[l]))
    ssum = jax.ops.segment_sum(h, batch, num_segments=NGRAPH)
    cnt = jax.ops.segment_sum(jnp.ones((n,), h.dtype), batch, num_segments=NGRAPH)
    pooled = ssum / jnp.maximum(cnt, 1.0)[:, None]
    m = jax.nn.relu(_bn(pooled @ Wm1, gm1, bm1))
    m = jax.nn.relu(_bn(m @ Wm2, gm2, bm2))
    z = jnp.concatenate([m, descriptors], axis=1)
    z = jax.nn.relu(z @ W2 + b2)
    z = _bn(z, g3, be3)
    return z @ Wout + bout
